# baseline (speedup 1.0000x reference)
import functools

import jax
import jax.numpy as jnp
from jax import lax
from jax.experimental import pallas as pl
from jax.experimental.pallas import tpu as pltpu
from jax.experimental.pallas import tpu_sc as plsc

_BATCH = 4096
_SEQ = 50
_D = 128
_NC = 2
_NS = 16
_NW = _NC * _NS
_SLICES = 4
_BS = _BATCH // _SLICES

_mesh = plsc.VectorSubcoreMesh(core_axis_name="c", subcore_axis_name="s")


def _make_slice_kernel(bs):
    bt = bs // _NW
    nb = 4
    nbuf = 4
    nchunk = bt // nb
    nstep = nchunk // nbuf
    assert nchunk % nbuf == 0

    @functools.partial(
        pl.kernel,
        mesh=_mesh,
        out_type=jax.ShapeDtypeStruct((bs, _SEQ, _D), jnp.float32),
        scratch_types=[
            pltpu.VMEM((bt, _SEQ), jnp.int32),
            pltpu.VMEM((nbuf, nb, _SEQ, _D), jnp.float32),
        ] + [pltpu.SemaphoreType.DMA] * (2 * nbuf),
    )
    def slice_kernel(idx_hbm, table_hbm, out_hbm, idx_v, rows_v, *sems):
        gsem = sems[:nbuf]
        ssem = sems[nbuf:]
        wid = lax.axis_index("s") * _NC + lax.axis_index("c")
        bbase = wid * bt
        pltpu.sync_copy(idx_hbm.at[pl.ds(bbase, bt)], idx_v)

        def g_start(c, b):
            for k in range(nb):
                j = c * nb + k
                pltpu.async_copy(table_hbm.at[idx_v.at[j]], rows_v.at[b, k],
                                 gsem[b])

        def g_wait(c, b):
            for k in range(nb):
                j = c * nb + k
                pltpu.make_async_copy(table_hbm.at[idx_v.at[j]],
                                      rows_v.at[b, k], gsem[b]).wait()

        def s_start(c, b):
            pltpu.async_copy(rows_v.at[b],
                             out_hbm.at[pl.ds(bbase + c * nb, nb)], ssem[b])

        def s_wait(c, b):
            pltpu.make_async_copy(rows_v.at[b],
                                  out_hbm.at[pl.ds(bbase + c * nb, nb)],
                                  ssem[b]).wait()

        for b in range(nbuf):
            g_start(b, b)

        def step(s, _):
            for b in range(nbuf):
                c = s * nbuf + b
                g_wait(c, b)
                s_start(c, b)
            for b in range(nbuf):
                c = s * nbuf + b
                s_wait(c, b)

                @pl.when(s < nstep - 1)
                def _():
                    g_start(c + nbuf, b)

            return 0

        lax.fori_loop(0, nstep, step, 0)

    return slice_kernel


_slice_kernel = _make_slice_kernel(_BS)


def kernel(x, table):
    outs = [_slice_kernel(x[i * _BS:(i + 1) * _BS], table)
            for i in range(_SLICES)]
    return jnp.concatenate(outs, axis=0)


# speedup vs baseline: 1.0433x; 1.0433x over previous
import functools

import jax
import jax.numpy as jnp
from jax import lax
from jax.experimental import pallas as pl
from jax.experimental.pallas import tpu as pltpu
from jax.experimental.pallas import tpu_sc as plsc

_BATCH = 4096
_SEQ = 50
_D = 128
_NC = 2
_NS = 16
_NW = _NC * _NS
_SLICES = 4
_BS = _BATCH // _SLICES

_mesh = plsc.VectorSubcoreMesh(core_axis_name="c", subcore_axis_name="s")


def _make_slice_kernel(bs):
    bt = bs // _NW
    nb = 4
    nbuf = 4
    nchunk = bt // nb
    nstep = nchunk // nbuf
    assert nchunk % nbuf == 0

    @functools.partial(
        pl.kernel,
        mesh=_mesh,
        out_type=jax.ShapeDtypeStruct((bs, _SEQ, _D), jnp.float32),
        scratch_types=[
            pltpu.VMEM((bt, _SEQ), jnp.int32),
            pltpu.VMEM((nbuf, nb, _SEQ, _D), jnp.float32),
        ] + [pltpu.SemaphoreType.DMA] * (2 * nbuf),
    )
    def slice_kernel(idx_hbm, table_hbm, out_hbm, idx_v, rows_v, *sems):
        gsem = sems[:nbuf]
        ssem = sems[nbuf:]
        wid = lax.axis_index("s") * _NC + lax.axis_index("c")
        bbase = wid * bt
        pltpu.sync_copy(idx_hbm.at[pl.ds(bbase, bt)], idx_v)

        def g_start(c, b):
            for k in range(nb):
                j = c * nb + k
                pltpu.async_copy(table_hbm.at[idx_v.at[j]], rows_v.at[b, k],
                                 gsem[b])

        def g_wait(c, b):
            for k in range(nb):
                j = c * nb + k
                pltpu.make_async_copy(table_hbm.at[idx_v.at[j]],
                                      rows_v.at[b, k], gsem[b]).wait()

        def s_start(c, b):
            pltpu.async_copy(rows_v.at[b],
                             out_hbm.at[pl.ds(bbase + c * nb, nb)], ssem[b])

        def s_wait(c, b):
            pltpu.make_async_copy(rows_v.at[b],
                                  out_hbm.at[pl.ds(bbase + c * nb, nb)],
                                  ssem[b]).wait()

        for b in range(nbuf):
            g_start(b, b)

        def step(s, _):
            for b in range(nbuf):
                c = s * nbuf + b
                g_wait(c, b)
                s_start(c, b)
            for b in range(nbuf):
                c = s * nbuf + b
                s_wait(c, b)

                @pl.when(s < nstep - 1)
                def _():
                    g_start(c + nbuf, b)

            return 0

        lax.fori_loop(0, nstep, step, 0)

    return slice_kernel


_slice_kernel = _make_slice_kernel(_BS)


def kernel(x, table):
    outs = [_slice_kernel(x[i * _BS:(i + 1) * _BS], table)
            for i in range(_SLICES)]
    out = jnp.pad(outs[0], ((0, _BATCH - _BS), (0, 0), (0, 0)))
    for i in range(1, _SLICES):
        out = lax.dynamic_update_slice(out, outs[i], (i * _BS, 0, 0))
    return out


# speedup vs baseline: 3.2124x; 3.0792x over previous
import functools

import jax
import jax.numpy as jnp
from jax import lax
from jax.experimental import pallas as pl
from jax.experimental.pallas import tpu as pltpu
from jax.experimental.pallas import tpu_sc as plsc

_BATCH = 4096
_SEQ = 50
_D = 128
_NC = 2
_NS = 16
_NW = _NC * _NS
_BT = _BATCH // _NW
_NBUF = 8
_NSTEP = _BT // _NBUF

_mesh = plsc.VectorSubcoreMesh(core_axis_name="c", subcore_axis_name="s")


@functools.partial(
    pl.kernel,
    mesh=_mesh,
    out_type=jax.ShapeDtypeStruct((_SEQ, _BATCH, _D), jnp.float32),
    scratch_types=[
        pltpu.VMEM((_BT, _SEQ), jnp.int32),
        pltpu.VMEM((_NBUF, _SEQ, _D), jnp.float32),
    ] + [pltpu.SemaphoreType.DMA] * (2 * _NBUF),
)
def _gather_kernel(idx_hbm, table_hbm, out_hbm, idx_v, rows_v, *sems):
    gsem = sems[:_NBUF]
    ssem = sems[_NBUF:]
    wid = lax.axis_index("s") * _NC + lax.axis_index("c")
    bbase = wid * _BT
    pltpu.sync_copy(idx_hbm.at[pl.ds(bbase, _BT)], idx_v)

    def g_start(j, b):
        pltpu.async_copy(table_hbm.at[idx_v.at[j]], rows_v.at[b], gsem[b])

    def g_wait(j, b):
        pltpu.make_async_copy(table_hbm.at[idx_v.at[j]], rows_v.at[b],
                              gsem[b]).wait()

    def s_start(j, b):
        pltpu.async_copy(rows_v.at[b], out_hbm.at[:, bbase + j], ssem[b])

    def s_wait(j, b):
        pltpu.make_async_copy(rows_v.at[b], out_hbm.at[:, bbase + j],
                              ssem[b]).wait()

    for b in range(_NBUF):
        g_start(b, b)

    def step(s, _):
        for b in range(_NBUF):
            j = s * _NBUF + b
            g_wait(j, b)
            s_start(j, b)
        for b in range(_NBUF):
            j = s * _NBUF + b
            s_wait(j, b)

            @pl.when(s < _NSTEP - 1)
            def _():
                g_start(j + _NBUF, b)

        return 0

    lax.fori_loop(0, _NSTEP, step, 0)


def kernel(x, table):
    out = _gather_kernel(x, table)
    return jnp.transpose(out, (1, 0, 2))
